# Initial kernel scaffold; baseline (speedup 1.0000x reference)
import jax
import jax.numpy as jnp
from jax.experimental import pallas as pl


def kernel(X, W):
    raise NotImplementedError("write your pallas kernel here")


# speedup vs baseline: 17.2529x; 17.2529x over previous
import functools

import jax
import jax.numpy as jnp
from jax import lax
from jax.experimental import pallas as pl
from jax.experimental.pallas import tpu as pltpu
from jax.experimental.pallas import tpu_sc as plsc

BATCH = 4096
SEQ = 50
K = 20
D = 32
B = BATCH * SEQ

NC = 2
NS = 16
NW = NC * NS
BPW = B // NW

CB = 64
RPC = CB * K
NSUB = RPC // 128
NCH = BPW // CB


def _bag_body(x_hbm, w_hbm, out_hbm,
              idx0, idx1, rows0, rows1, outb, sem0, sem1):
    wid = lax.axis_index("s") * NC + lax.axis_index("c")
    idx_bufs = (idx0, idx1)
    rows_bufs = (rows0, rows1)
    sems = (sem0, sem1)

    def start(g, slot):
        pltpu.sync_copy(x_hbm.at[wid, g], idx_bufs[slot])
        for t in range(NSUB):
            pltpu.async_copy(
                w_hbm.at[idx_bufs[slot].at[t]],
                rows_bufs[slot].at[pl.ds(t * 128, 128)],
                sems[slot])

    def finish(g, slot):
        rows = rows_bufs[slot]
        for t in range(NSUB):
            pltpu.make_async_copy(
                w_hbm.at[idx_bufs[slot].at[t]],
                rows.at[pl.ds(t * 128, 128)],
                sems[slot]).wait()

        def bag(b, carry):
            r = b * K
            acc0 = rows[r, 0:16]
            acc1 = rows[r, 16:32]
            for j in range(1, K):
                acc0 = acc0 + rows[r + j, 0:16]
                acc1 = acc1 + rows[r + j, 16:32]
            outb[b, 0:16] = acc0
            outb[b, 16:32] = acc1
            return carry

        lax.fori_loop(0, CB, bag, 0, unroll=2)
        pltpu.sync_copy(outb, out_hbm.at[pl.ds(wid * BPW + g * CB, CB)])

    start(0, 0)

    def pipeline(g, carry):
        @pl.when(g + 1 < NCH)
        def _():
            start(g + 1, 1)
        finish(g, 0)

        @pl.when(g + 1 < NCH)
        def _():
            @pl.when(g + 2 < NCH)
            def _():
                start(g + 2, 0)
            finish(g + 1, 1)
        return carry

    lax.fori_loop(0, NCH // 2, lambda i, c: pipeline(i * 2, c), 0)


@jax.jit
def _bag_embedding(x_flat, w):
    mesh = plsc.VectorSubcoreMesh(core_axis_name="c", subcore_axis_name="s",
                                  num_cores=NC, num_subcores=NS)
    run = pl.kernel(
        _bag_body,
        out_type=jax.ShapeDtypeStruct((B, D), jnp.float32),
        mesh=mesh,
        scratch_types=[
            pltpu.VMEM((NSUB, 128), jnp.int32),
            pltpu.VMEM((NSUB, 128), jnp.int32),
            pltpu.VMEM((RPC, D), jnp.float32),
            pltpu.VMEM((RPC, D), jnp.float32),
            pltpu.VMEM((CB, D), jnp.float32),
            pltpu.SemaphoreType.DMA,
            pltpu.SemaphoreType.DMA,
        ],
        compiler_params=pltpu.CompilerParams(use_tc_tiling_on_sc=False),
    )
    return run(x_flat, w)


def kernel(X, W):
    x_flat = X.reshape(NW, NCH, NSUB, 128)
    out = _bag_embedding(x_flat, W)
    return out.reshape(BATCH, SEQ, D)


# speedup vs baseline: 21.5041x; 1.2464x over previous
import jax
import jax.numpy as jnp
from jax import lax
from jax.experimental import pallas as pl
from jax.experimental.pallas import tpu as pltpu
from jax.experimental.pallas import tpu_sc as plsc

BATCH = 4096
SEQ = 50
K = 20
D = 32
B = BATCH * SEQ

NC = 2
NS = 16
NW = NC * NS

CB = 64
RPC = CB * K
NCHUNK = SEQ * (BATCH // CB)
CPW = NCHUNK // NW


def _bag_body(x_hbm, w_hbm, out_hbm,
              idx0, idx1, rows0, rows1, outb, sem0, sem1):
    wid = lax.axis_index("s") * NC + lax.axis_index("c")
    idx_bufs = (idx0, idx1)
    rows_bufs = (rows0, rows1)
    sems = (sem0, sem1)


    def start(c, slot):
        l = c // (BATCH // CB)
        b0 = (c % (BATCH // CB)) * CB
        idx = idx_bufs[slot]
        pltpu.sync_copy(x_hbm.at[:, pl.ds(l * BATCH + b0, CB)], idx)
        for t in range(K):
            pltpu.async_copy(
                w_hbm.at[idx_bufs[slot].at[t]],
                rows_bufs[slot].at[pl.ds(t * CB, CB)],
                sems[slot])

    def finish(c, slot):
        rows = rows_bufs[slot]
        for t in range(K):
            pltpu.make_async_copy(
                w_hbm.at[idx_bufs[slot].at[t]],
                rows.at[pl.ds(t * CB, CB)],
                sems[slot]).wait()

        def bag(b, carry):
            acc0 = rows[b, 0:16]
            acc1 = rows[b, 16:32]
            for j in range(1, K):
                acc0 = acc0 + rows[j * CB + b, 0:16]
                acc1 = acc1 + rows[j * CB + b, 16:32]
            iota = lax.iota(jnp.int32, 16)
            col = jnp.zeros((16,), jnp.int32) + b
            plsc.store_scatter(outb, [iota, col], acc0)
            plsc.store_scatter(outb, [iota + 16, col], acc1)
            return carry

        lax.fori_loop(0, CB, bag, 0, unroll=2)
        l = c // (BATCH // CB)
        b0 = (c % (BATCH // CB)) * CB
        pltpu.sync_copy(outb, out_hbm.at[l, :, pl.ds(b0, CB)])

    c_base = wid * CPW
    start(c_base, 0)

    def pipeline(g, carry):
        @pl.when(g + 1 < CPW)
        def _():
            start(c_base + g + 1, 1)
        finish(c_base + g, 0)

        @pl.when(g + 1 < CPW)
        def _():
            @pl.when(g + 2 < CPW)
            def _():
                start(c_base + g + 2, 0)
            finish(c_base + g + 1, 1)
        return carry

    lax.fori_loop(0, CPW // 2, lambda i, c: pipeline(i * 2, c), 0)


@jax.jit
def _bag_embedding(x_kmajor, w):
    mesh = plsc.VectorSubcoreMesh(core_axis_name="c", subcore_axis_name="s",
                                  num_cores=NC, num_subcores=NS)
    run = pl.kernel(
        _bag_body,
        out_type=jax.ShapeDtypeStruct((SEQ, D, BATCH), jnp.float32),
        mesh=mesh,
        scratch_types=[
            pltpu.VMEM((K, CB), jnp.int32),
            pltpu.VMEM((K, CB), jnp.int32),
            pltpu.VMEM((RPC, D), jnp.float32),
            pltpu.VMEM((RPC, D), jnp.float32),
            pltpu.VMEM((D, CB), jnp.float32),
            pltpu.SemaphoreType.DMA,
            pltpu.SemaphoreType.DMA,
        ],
        compiler_params=pltpu.CompilerParams(use_tc_tiling_on_sc=False,
                                             needs_layout_passes=False),
    )
    return run(x_kmajor, w)


def kernel(X, W):
    x_kmajor = jnp.transpose(X, (2, 1, 0)).reshape(K, B)
    out = _bag_embedding(x_kmajor, W)
    return out.transpose(2, 0, 1)
